# Initial kernel scaffold; baseline (speedup 1.0000x reference)
import functools

import jax
import jax.numpy as jnp
from jax.experimental import pallas as pl
from jax.experimental.pallas import tpu as pltpu

_NUM_SKILL = 8
_TOP_K = 2
_TI = 1024


def _skill_kernel(idx_ref, vals_ref, x_ref, wg_ref, wu_ref, wd_ref, out_ref):
    b = pl.program_id(0)
    k = pl.program_id(2)
    i = pl.program_id(3)
    x2 = x_ref[0]
    g = jnp.dot(x2, wg_ref[0], preferred_element_type=jnp.float32)
    u = jnp.dot(x2, wu_ref[0], preferred_element_type=jnp.float32)
    h = jax.nn.gelu(g, approximate=True) * u
    contrib = jnp.dot(h, wd_ref[0], preferred_element_type=jnp.float32)
    contrib = contrib * vals_ref[b, k]

    @pl.when((k == 0) & (i == 0))
    def _init():
        out_ref[0] = contrib

    @pl.when((k > 0) | (i > 0))
    def _acc():
        out_ref[0] = out_ref[0] + contrib


def _shared_kernel(x_ref, wg_ref, wu_ref, wd_ref, part_ref, out_ref):
    e = pl.program_id(2)
    i = pl.program_id(3)
    x2 = x_ref[0]
    g = jnp.dot(x2, wg_ref[0], preferred_element_type=jnp.float32)
    u = jnp.dot(x2, wu_ref[0], preferred_element_type=jnp.float32)
    h = jax.nn.gelu(g, approximate=True) * u
    contrib = jnp.dot(h, wd_ref[0], preferred_element_type=jnp.float32)

    @pl.when((e == 0) & (i == 0))
    def _init():
        out_ref[0] = part_ref[0] + contrib

    @pl.when((e > 0) | (i > 0))
    def _acc():
        out_ref[0] = out_ref[0] + contrib


@jax.jit
def kernel(x, router_logits, skill_gate, skill_up, skill_down, shared_gate, shared_up, shared_down):
    B, S, H = x.shape
    E, _, I = skill_gate.shape
    E_sh = shared_gate.shape[0]
    n_i = I // _TI

    rw = jax.nn.softmax(router_logits.astype(jnp.float32), axis=-1)
    vals, idx = jax.lax.top_k(rw, _TOP_K)
    vals = (vals / jnp.sum(vals, axis=-1, keepdims=True)).astype(x.dtype)

    grid = (B, 1, _TOP_K, n_i)

    skill_out = pl.pallas_call(
        _skill_kernel,
        grid_spec=pltpu.PrefetchScalarGridSpec(
            num_scalar_prefetch=2,
            grid=grid,
            in_specs=[
                pl.BlockSpec((1, S, H), lambda b, s, k, i, idx, vals: (b, s, 0)),
                pl.BlockSpec((1, H, _TI), lambda b, s, k, i, idx, vals: (idx[b, k], 0, i)),
                pl.BlockSpec((1, H, _TI), lambda b, s, k, i, idx, vals: (idx[b, k], 0, i)),
                pl.BlockSpec((1, _TI, H), lambda b, s, k, i, idx, vals: (idx[b, k], i, 0)),
            ],
            out_specs=pl.BlockSpec((1, S, H), lambda b, s, k, i, idx, vals: (b, s, 0)),
        ),
        out_shape=jax.ShapeDtypeStruct((B, S, H), x.dtype),
    )(idx, vals, x, skill_gate, skill_up, skill_down)

    out = pl.pallas_call(
        _shared_kernel,
        grid=(B, 1, E_sh, n_i),
        in_specs=[
            pl.BlockSpec((1, S, H), lambda b, s, e, i: (b, s, 0)),
            pl.BlockSpec((1, H, _TI), lambda b, s, e, i: (e, 0, i)),
            pl.BlockSpec((1, H, _TI), lambda b, s, e, i: (e, 0, i)),
            pl.BlockSpec((1, _TI, H), lambda b, s, e, i: (e, i, 0)),
            pl.BlockSpec((1, S, H), lambda b, s, e, i: (b, s, 0)),
        ],
        out_specs=pl.BlockSpec((1, S, H), lambda b, s, e, i: (b, s, 0)),
        out_shape=jax.ShapeDtypeStruct((B, S, H), x.dtype),
    )(x, shared_gate, shared_up, shared_down, skill_out)

    return out


# speedup vs baseline: 3.5602x; 3.5602x over previous
import functools

import jax
import jax.numpy as jnp
from jax.experimental import pallas as pl
from jax.experimental.pallas import tpu as pltpu

_NUM_SKILL = 8
_TOP_K = 2
_TI = 512
_TS_SH = 1024


def _skill_kernel(idx_ref, vals_ref, x_ref, wg_ref, wu_ref, wd_ref, out_ref):
    b = pl.program_id(0)
    k = pl.program_id(2)
    i = pl.program_id(3)
    x2 = x_ref[0]
    g = jnp.dot(x2, wg_ref[0], preferred_element_type=jnp.float32)
    u = jnp.dot(x2, wu_ref[0], preferred_element_type=jnp.float32)
    h = jax.nn.gelu(g, approximate=True) * u
    contrib = jnp.dot(h, wd_ref[0], preferred_element_type=jnp.float32)
    contrib = contrib * vals_ref[b, k]

    @pl.when((k == 0) & (i == 0))
    def _init():
        out_ref[0] = contrib

    @pl.when((k > 0) | (i > 0))
    def _acc():
        out_ref[0] = out_ref[0] + contrib


def _shared_kernel(x_ref, wg_ref, wu_ref, wd_ref, part_ref, out_ref):
    e = pl.program_id(2)
    i = pl.program_id(3)
    x2 = x_ref[0]
    g = jnp.dot(x2, wg_ref[0], preferred_element_type=jnp.float32)
    u = jnp.dot(x2, wu_ref[0], preferred_element_type=jnp.float32)
    h = jax.nn.gelu(g, approximate=True) * u
    contrib = jnp.dot(h, wd_ref[0], preferred_element_type=jnp.float32)

    @pl.when((e == 0) & (i == 0))
    def _init():
        out_ref[0] = part_ref[0] + contrib

    @pl.when((e > 0) | (i > 0))
    def _acc():
        out_ref[0] = out_ref[0] + contrib


@jax.jit
def kernel(x, router_logits, skill_gate, skill_up, skill_down, shared_gate, shared_up, shared_down):
    B, S, H = x.shape
    E, _, I = skill_gate.shape
    E_sh = shared_gate.shape[0]
    n_i = I // _TI

    rw = jax.nn.softmax(router_logits.astype(jnp.float32), axis=-1)
    vals, idx = jax.lax.top_k(rw, _TOP_K)
    vals = (vals / jnp.sum(vals, axis=-1, keepdims=True)).astype(x.dtype)

    grid = (B, 1, _TOP_K, n_i)

    skill_out = pl.pallas_call(
        _skill_kernel,
        grid_spec=pltpu.PrefetchScalarGridSpec(
            num_scalar_prefetch=2,
            grid=grid,
            in_specs=[
                pl.BlockSpec((1, S, H), lambda b, s, k, i, idx, vals: (b, s, 0)),
                pl.BlockSpec((1, H, _TI), lambda b, s, k, i, idx, vals: (idx[b, k], 0, i)),
                pl.BlockSpec((1, H, _TI), lambda b, s, k, i, idx, vals: (idx[b, k], 0, i)),
                pl.BlockSpec((1, _TI, H), lambda b, s, k, i, idx, vals: (idx[b, k], i, 0)),
            ],
            out_specs=pl.BlockSpec((1, S, H), lambda b, s, k, i, idx, vals: (b, s, 0)),
        ),
        out_shape=jax.ShapeDtypeStruct((B, S, H), x.dtype),
    )(idx, vals, x, skill_gate, skill_up, skill_down)

    out = pl.pallas_call(
        _shared_kernel,
        grid=(B, S // _TS_SH, E_sh, n_i),
        in_specs=[
            pl.BlockSpec((1, _TS_SH, H), lambda b, s, e, i: (b, s, 0)),
            pl.BlockSpec((1, H, _TI), lambda b, s, e, i: (e, 0, i)),
            pl.BlockSpec((1, H, _TI), lambda b, s, e, i: (e, 0, i)),
            pl.BlockSpec((1, _TI, H), lambda b, s, e, i: (e, i, 0)),
            pl.BlockSpec((1, _TS_SH, H), lambda b, s, e, i: (b, s, 0)),
        ],
        out_specs=pl.BlockSpec((1, _TS_SH, H), lambda b, s, e, i: (b, s, 0)),
        out_shape=jax.ShapeDtypeStruct((B, S, H), x.dtype),
    )(x, shared_gate, shared_up, shared_down, skill_out)

    return out
